# baseline (speedup 1.0000x reference)
import functools

import jax
import jax.numpy as jnp
from jax import lax
from jax.experimental import pallas as pl
from jax.experimental.pallas import tpu as pltpu
from jax.experimental.pallas import tpu_sc as plsc


def kernel(x, pos_emb):
    batch, seq, d = x.shape
    V, D = pos_emb.shape
    info = plsc.get_sparse_core_info()
    nc = info.num_cores
    rows_per_c = V // nc
    n_chunks = 4
    crows = rows_per_c // n_chunks
    mesh = plsc.ScalarSubcoreMesh(axis_name="c", num_cores=nc)

    @functools.partial(
        pl.kernel,
        mesh=mesh,
        out_type=jax.ShapeDtypeStruct((batch * V, D), pos_emb.dtype),
        scratch_types=[
            pltpu.VMEM_SHARED((rows_per_c, D), jnp.float32),
            pltpu.SemaphoreType.DMA,
            pltpu.SemaphoreType.DMA,
        ],
    )
    def bcast(pe_hbm, out_hbm, buf, rsem, wsem):
        cid = lax.axis_index("c")
        base = cid * rows_per_c
        reads = [
            pltpu.async_copy(
                pe_hbm.at[pl.ds(base + c * crows, crows)],
                buf.at[pl.ds(c * crows, crows)],
                rsem,
            )
            for c in range(n_chunks)
        ]
        writes = []
        for c in range(n_chunks):
            reads[c].wait()
            writes += [
                pltpu.async_copy(
                    buf.at[pl.ds(c * crows, crows)],
                    out_hbm.at[pl.ds(b * V + base + c * crows, crows)],
                    wsem,
                )
                for b in range(batch)
            ]
        for w in writes:
            w.wait()

    return bcast(pos_emb).reshape(batch, V, D)


# speedup vs baseline: 1.1649x; 1.1649x over previous
import functools

import jax
import jax.numpy as jnp
from jax import lax
from jax.experimental import pallas as pl
from jax.experimental.pallas import tpu as pltpu
from jax.experimental.pallas import tpu_sc as plsc


def kernel(x, pos_emb):
    batch, seq, d = x.shape
    V, D = pos_emb.shape
    info = plsc.get_sparse_core_info()
    nc, ns = info.num_cores, info.num_subcores
    nw = nc * ns
    rows_per_w = V // nw
    n_chunks = 4
    crows = rows_per_w // n_chunks
    mesh = plsc.VectorSubcoreMesh(core_axis_name="c", subcore_axis_name="s")

    @functools.partial(
        pl.kernel,
        mesh=mesh,
        out_type=jax.ShapeDtypeStruct((batch * V, D), pos_emb.dtype),
        scratch_types=[
            pltpu.VMEM((rows_per_w, D), jnp.float32),
            pltpu.SemaphoreType.DMA,
            pltpu.SemaphoreType.DMA,
        ],
    )
    def bcast(pe_hbm, out_hbm, buf, rsem, wsem):
        wid = lax.axis_index("s") * nc + lax.axis_index("c")
        base = wid * rows_per_w
        reads = [
            pltpu.async_copy(
                pe_hbm.at[pl.ds(base + c * crows, crows)],
                buf.at[pl.ds(c * crows, crows)],
                rsem,
            )
            for c in range(n_chunks)
        ]
        writes = []
        for c in range(n_chunks):
            reads[c].wait()
            writes += [
                pltpu.async_copy(
                    buf.at[pl.ds(c * crows, crows)],
                    out_hbm.at[pl.ds(b * V + base + c * crows, crows)],
                    wsem,
                )
                for b in range(batch)
            ]
        for w in writes:
            w.wait()

    return bcast(pos_emb).reshape(batch, V, D)
